# baseline (speedup 1.0000x reference)
import jax
import jax.numpy as jnp
from jax.experimental import pallas as pl
from jax.experimental.pallas import tpu as pltpu

_B, _S, _DIN, _DOUT, _E = 2, 4096, 2048, 2048, 8
_R = 64
_RP = _R // _E
_SCALING = 16.0 / _R

_ST = 512
_OT = 512


def _moe_lora_kernel(emb_ref, rw_ref, a_ref, b2_ref, w_ref, x_ref,
                     o_ref, weff_ref):
    @pl.when(pl.program_id(1) == 0)
    def _():
        logits = jnp.sum(rw_ref[...] * emb_ref[...], axis=1, keepdims=True)
        m = jnp.max(logits)
        p = jnp.exp(logits - m)
        probs = p / jnp.sum(p)
        i_idx = jax.lax.broadcasted_iota(jnp.int32, (_R, _E), 0) // _RP
        e_idx = jax.lax.broadcasted_iota(jnp.int32, (_R, _E), 1)
        onehot = (i_idx == e_idx).astype(jnp.float32)
        s64 = jax.lax.dot_general(
            onehot, probs, (((1,), (0,)), ((), ())),
            preferred_element_type=jnp.float32)
        a_scaled = a_ref[...] * (s64 * _SCALING)
        delta = jax.lax.dot_general(
            b2_ref[...], a_scaled, (((1,), (0,)), ((), ())),
            preferred_element_type=jnp.float32)
        weff_ref[...] = w_ref[...] + delta

    o_ref[...] = jax.lax.dot_general(
        x_ref[...], weff_ref[...], (((1,), (1,)), ((), ())),
        preferred_element_type=jnp.float32)


def kernel(x, task_ids, base_W, loraA, loraB, expert_emb, router_W):
    del task_ids
    xf = x.reshape(_B * _S, _DIN)
    a_all = loraA.reshape(_R, _DIN)
    b2 = loraB.transpose(1, 0, 2).reshape(_DOUT, _R)
    grid = (_DOUT // _OT, (_B * _S) // _ST)
    out = pl.pallas_call(
        _moe_lora_kernel,
        grid=grid,
        in_specs=[
            pl.BlockSpec((1, _DOUT), lambda o, s: (0, 0)),
            pl.BlockSpec((_E, _DOUT), lambda o, s: (0, 0)),
            pl.BlockSpec((_R, _DIN), lambda o, s: (0, 0)),
            pl.BlockSpec((_OT, _R), lambda o, s: (o, 0)),
            pl.BlockSpec((_OT, _DIN), lambda o, s: (o, 0)),
            pl.BlockSpec((_ST, _DIN), lambda o, s: (s, 0)),
        ],
        out_specs=pl.BlockSpec((_ST, _OT), lambda o, s: (s, o)),
        out_shape=jax.ShapeDtypeStruct((_B * _S, _DOUT), jnp.float32),
        scratch_shapes=[pltpu.VMEM((_OT, _DIN), jnp.float32)],
        compiler_params=pltpu.CompilerParams(
            dimension_semantics=("parallel", "arbitrary")),
    )(expert_emb, router_W, a_all, b2, base_W, xf)
    return out.reshape(_B, _S, _DOUT)


# speedup vs baseline: 1.6346x; 1.6346x over previous
import jax
import jax.numpy as jnp
from jax.experimental import pallas as pl
from jax.experimental.pallas import tpu as pltpu

_B, _S, _DIN, _DOUT, _E = 2, 4096, 2048, 2048, 8
_R = 64
_RP = _R // _E
_SCALING = 16.0 / _R

_ST = 512
_OT = 2048


def _moe_lora_kernel(emb_ref, rw_ref, a_ref, b2_ref, w_ref, x_ref,
                     o_ref, weff_ref):
    @pl.when(pl.program_id(1) == 0)
    def _():
        logits = jnp.sum(rw_ref[...] * emb_ref[...], axis=1, keepdims=True)
        m = jnp.max(logits)
        p = jnp.exp(logits - m)
        probs = p / jnp.sum(p)
        i_idx = jax.lax.broadcasted_iota(jnp.int32, (_R, _E), 0) // _RP
        e_idx = jax.lax.broadcasted_iota(jnp.int32, (_R, _E), 1)
        onehot = (i_idx == e_idx).astype(jnp.float32)
        s64 = jax.lax.dot_general(
            onehot, probs, (((1,), (0,)), ((), ())),
            preferred_element_type=jnp.float32)
        a_scaled = a_ref[...] * (s64 * _SCALING)
        delta = jax.lax.dot_general(
            b2_ref[...], a_scaled, (((1,), (0,)), ((), ())),
            preferred_element_type=jnp.float32)
        weff_ref[...] = w_ref[...] + delta

    o_ref[...] = jax.lax.dot_general(
        x_ref[...], weff_ref[...], (((1,), (1,)), ((), ())),
        preferred_element_type=jnp.float32)


def kernel(x, task_ids, base_W, loraA, loraB, expert_emb, router_W):
    del task_ids
    xf = x.reshape(_B * _S, _DIN)
    a_all = loraA.reshape(_R, _DIN)
    b2 = loraB.transpose(1, 0, 2).reshape(_DOUT, _R)
    grid = (_DOUT // _OT, (_B * _S) // _ST)
    out = pl.pallas_call(
        _moe_lora_kernel,
        grid=grid,
        in_specs=[
            pl.BlockSpec((1, _DOUT), lambda o, s: (0, 0)),
            pl.BlockSpec((_E, _DOUT), lambda o, s: (0, 0)),
            pl.BlockSpec((_R, _DIN), lambda o, s: (0, 0)),
            pl.BlockSpec((_OT, _R), lambda o, s: (o, 0)),
            pl.BlockSpec((_OT, _DIN), lambda o, s: (o, 0)),
            pl.BlockSpec((_ST, _DIN), lambda o, s: (s, 0)),
        ],
        out_specs=pl.BlockSpec((_ST, _OT), lambda o, s: (s, o)),
        out_shape=jax.ShapeDtypeStruct((_B * _S, _DOUT), jnp.float32),
        scratch_shapes=[pltpu.VMEM((_OT, _DIN), jnp.float32)],
        compiler_params=pltpu.CompilerParams(
            dimension_semantics=("parallel", "arbitrary")),
    )(expert_emb, router_W, a_all, b2, base_W, xf)
    return out.reshape(_B, _S, _DOUT)
